# baseline (speedup 1.0000x reference)
import functools

import jax
import jax.numpy as jnp
from jax.experimental import pallas as pl

_NB = 2000


def _attn_topk_block(x_ref, wt_ref, b_ref, a2_ref, o_ref):
    P, NB, D = x_ref.shape
    x = x_ref[...]
    xf = x.reshape(P * NB, D)
    z = jnp.dot(xf, wt_ref[...], preferred_element_type=jnp.float32)
    h = jnp.tanh(z + b_ref[...])
    h3 = h.reshape(P, NB, D)
    a2 = a2_ref[...]
    dims = (((1,), (1,)), ((), ()))
    l = jnp.concatenate(
        [
            jax.lax.dot_general(a2, h3[p], dims,
                                preferred_element_type=jnp.float32)
            for p in range(P)
        ],
        axis=0,
    )

    pidx = jax.lax.broadcasted_iota(jnp.int32, l.shape, 0)
    m1 = jnp.max(l, axis=0, keepdims=True)
    idx1 = jnp.min(jnp.where(l == m1, pidx, P), axis=0, keepdims=True)
    sel1 = pidx == idx1
    l2 = jnp.where(sel1, -1e30, l)
    m2 = jnp.max(l2, axis=0, keepdims=True)
    idx2 = jnp.min(jnp.where(l2 == m2, pidx, P), axis=0, keepdims=True)
    sel2 = pidx == idx2

    e = jnp.exp(l - m1)
    denom = jnp.sum(e, axis=0, keepdims=True)
    w = e / denom
    wsel = jnp.where(sel1 | sel2, w, jnp.float32(0.0))

    ones_row = jnp.full((1, D), 1.0, jnp.float32)
    acc = jnp.zeros((NB, D), jnp.float32)
    for p in range(P):
        wsel_t = jnp.transpose(wsel[p:p + 1])
        wx = jax.lax.dot_general(
            wsel_t, ones_row, (((1,), (0,)), ((), ())),
            preferred_element_type=jnp.float32,
        )
        acc = acc + wx * x[p]
    o_ref[...] = acc


@functools.partial(jax.jit, static_argnames=("interpret",))
def kernel(semantic_embeddings, attnVec, fc_w, fc_b, interpret=False):
    P, N, D = semantic_embeddings.shape
    NB = _NB
    a2 = attnVec.reshape(1, D)
    b2 = fc_b.reshape(1, D)

    grid = (N // NB,)
    out = pl.pallas_call(
        _attn_topk_block,
        grid=grid,
        in_specs=[
            pl.BlockSpec((P, NB, D), lambda i: (0, i, 0)),
            pl.BlockSpec((D, D), lambda i: (0, 0)),
            pl.BlockSpec((1, D), lambda i: (0, 0)),
            pl.BlockSpec((1, D), lambda i: (0, 0)),
        ],
        out_specs=pl.BlockSpec((NB, D), lambda i: (i, 0)),
        out_shape=jax.ShapeDtypeStruct((N, D), jnp.float32),
        interpret=interpret,
    )(semantic_embeddings, fc_w.T, b2, a2)
    return out


# speedup vs baseline: 1.0224x; 1.0224x over previous
import functools

import jax
import jax.numpy as jnp
from jax.experimental import pallas as pl

_NB = 4000


def _attn_topk_block(x_ref, wt_ref, b_ref, a2_ref, o_ref):
    P, NB, D = x_ref.shape
    x = x_ref[...]
    xf = x.reshape(P * NB, D)
    z = jnp.dot(xf, wt_ref[...], preferred_element_type=jnp.float32)
    h = jnp.tanh(z + b_ref[...])
    h3 = h.reshape(P, NB, D)
    a2 = a2_ref[...]
    dims = (((1,), (1,)), ((), ()))
    l = jnp.concatenate(
        [
            jax.lax.dot_general(a2, h3[p], dims,
                                preferred_element_type=jnp.float32)
            for p in range(P)
        ],
        axis=0,
    )

    pidx = jax.lax.broadcasted_iota(jnp.int32, l.shape, 0)
    m1 = jnp.max(l, axis=0, keepdims=True)
    idx1 = jnp.min(jnp.where(l == m1, pidx, P), axis=0, keepdims=True)
    sel1 = pidx == idx1
    l2 = jnp.where(sel1, -1e30, l)
    m2 = jnp.max(l2, axis=0, keepdims=True)
    idx2 = jnp.min(jnp.where(l2 == m2, pidx, P), axis=0, keepdims=True)
    sel2 = pidx == idx2

    e = jnp.exp(l - m1)
    denom = jnp.sum(e, axis=0, keepdims=True)
    w = e / denom
    wsel = jnp.where(sel1 | sel2, w, jnp.float32(0.0))

    ones_row = jnp.full((1, D), 1.0, jnp.float32)
    acc = jnp.zeros((NB, D), jnp.float32)
    for p in range(P):
        wsel_t = jnp.transpose(wsel[p:p + 1])
        wx = jax.lax.dot_general(
            wsel_t, ones_row, (((1,), (0,)), ((), ())),
            preferred_element_type=jnp.float32,
        )
        acc = acc + wx * x[p]
    o_ref[...] = acc


@functools.partial(jax.jit, static_argnames=("interpret",))
def kernel(semantic_embeddings, attnVec, fc_w, fc_b, interpret=False):
    P, N, D = semantic_embeddings.shape
    NB = _NB
    a2 = attnVec.reshape(1, D)
    b2 = fc_b.reshape(1, D)

    grid = (N // NB,)
    out = pl.pallas_call(
        _attn_topk_block,
        grid=grid,
        in_specs=[
            pl.BlockSpec((P, NB, D), lambda i: (0, i, 0)),
            pl.BlockSpec((D, D), lambda i: (0, 0)),
            pl.BlockSpec((1, D), lambda i: (0, 0)),
            pl.BlockSpec((1, D), lambda i: (0, 0)),
        ],
        out_specs=pl.BlockSpec((NB, D), lambda i: (i, 0)),
        out_shape=jax.ShapeDtypeStruct((N, D), jnp.float32),
        interpret=interpret,
    )(semantic_embeddings, fc_w.T, b2, a2)
    return out
